# baseline (speedup 1.0000x reference)
import functools

import jax
import jax.numpy as jnp
from jax.experimental import pallas as pl
from jax.experimental.pallas import tpu as pltpu

B = 32
N = 128
D = 128
G = 2 * B
SIGMA = 2.5


def _mspnet_kernel(ca_ref, cb_ref, feats_ref, w1_ref, b1_ref, w2_ref, b2_ref,
                   wt1_ref, bt1_ref, wt2_ref, bt2_ref, out_ref, pooled):
    g = pl.program_id(0)

    ca = ca_ref[0]
    cb = cb_ref[0]
    d2 = jnp.zeros((N, N), jnp.float32)
    for k in range(3):
        dk = ca[:, k:k + 1] - cb[k:k + 1, :]
        d2 = d2 + dk * dk
    dist = jnp.sqrt(d2 + 1e-12)
    a = jnp.exp(dist * (-1.0 / SIGMA))
    row = jax.lax.broadcasted_iota(jnp.int32, (N, N), 0)
    col = jax.lax.broadcasted_iota(jnp.int32, (N, N), 1)
    a = jnp.where(row == col, 1.0, a)

    dinv_c = jax.lax.rsqrt(jnp.sum(a, axis=1, keepdims=True))
    dinv_r = jax.lax.rsqrt(jnp.sum(a, axis=0, keepdims=True))
    an = a * dinv_c * dinv_r

    x = feats_ref[0]
    h = jnp.dot(an, x, preferred_element_type=jnp.float32)
    h = jnp.dot(h, w1_ref[...], preferred_element_type=jnp.float32)
    h = jnp.maximum(h + b1_ref[...], 0.0)
    h = jnp.dot(an, h, preferred_element_type=jnp.float32)
    h = jnp.dot(h, w2_ref[...], preferred_element_type=jnp.float32)
    h = jnp.maximum(h + b2_ref[...], 0.0)
    pooled[pl.ds(g, 1), :] = jnp.max(h, axis=0, keepdims=True)

    @pl.when(g == G - 1)
    def _():
        xo = pooled[0:B, :]
        xm = pooled[B:G, :]
        t = (jnp.dot(xo, wt1_ref[0:D, :], preferred_element_type=jnp.float32)
             + jnp.dot(xm, wt1_ref[D:2 * D, :], preferred_element_type=jnp.float32)
             + bt1_ref[...])
        t = jnp.maximum(t, 0.0)
        out_ref[...] = (jnp.sum(t * wt2_ref[...], axis=1, keepdims=True)
                        + bt2_ref[...])


@functools.partial(jax.jit, static_argnames=("interpret",))
def kernel(coords_orig, feats_orig, coords_mut, feats_mut,
           W1, b1, W2, b2, Wt1, bt1, Wt2, bt2, interpret=False):
    coords = jnp.concatenate([coords_orig, coords_mut], axis=0)
    ca = jnp.pad(coords, ((0, 0), (0, 0), (0, 5)))
    cb = jnp.transpose(ca, (0, 2, 1))
    feats = jnp.concatenate([feats_orig, feats_mut], axis=0)

    const = lambda g: (0, 0)
    out = pl.pallas_call(
        _mspnet_kernel,
        grid=(G,),
        in_specs=[
            pl.BlockSpec((1, N, 8), lambda g: (g, 0, 0)),
            pl.BlockSpec((1, 8, N), lambda g: (g, 0, 0)),
            pl.BlockSpec((1, N, D), lambda g: (g, 0, 0)),
            pl.BlockSpec((D, D), const),
            pl.BlockSpec((1, D), const),
            pl.BlockSpec((D, D), const),
            pl.BlockSpec((1, D), const),
            pl.BlockSpec((2 * D, D), const),
            pl.BlockSpec((1, D), const),
            pl.BlockSpec((1, D), const),
            pl.BlockSpec((1, 1), const),
        ],
        out_specs=pl.BlockSpec((B, 1), const),
        out_shape=jax.ShapeDtypeStruct((B, 1), jnp.float32),
        scratch_shapes=[pltpu.VMEM((G, D), jnp.float32)],
        interpret=interpret,
    )(ca, cb, feats, W1, b1[None, :], W2, b2[None, :],
      Wt1, bt1[None, :], Wt2[:, 0][None, :], bt2.reshape(1, 1))
    return out


# speedup vs baseline: 2.3104x; 2.3104x over previous
import functools

import jax
import jax.numpy as jnp
from jax.experimental import pallas as pl
from jax.experimental.pallas import tpu as pltpu

B = 32
N = 128
D = 128
G = 2 * B
GP = 8
STEPS = G // GP
SIGMA = 2.5


def _mspnet_kernel(ca_ref, cb_ref, feats_ref, w1_ref, b1_ref, w2_ref, b2_ref,
                   wt1_ref, bt1_ref, wt2_ref, bt2_ref, out_ref, pooled):
    step = pl.program_id(0)

    row = jax.lax.broadcasted_iota(jnp.int32, (N, N), 0)
    col = jax.lax.broadcasted_iota(jnp.int32, (N, N), 1)

    ans = []
    for i in range(GP):
        ca = ca_ref[i]
        cb = cb_ref[i]
        d2 = jnp.zeros((N, N), jnp.float32)
        for k in range(3):
            dk = ca[:, k:k + 1] - cb[k:k + 1, :]
            d2 = d2 + dk * dk
        dist = jnp.sqrt(d2 + 1e-12)
        a = jnp.exp(dist * (-1.0 / SIGMA))
        a = jnp.where(row == col, 1.0, a)
        dinv_c = jax.lax.rsqrt(jnp.sum(a, axis=1, keepdims=True))
        dinv_r = jax.lax.rsqrt(jnp.sum(a, axis=0, keepdims=True))
        ans.append(a * dinv_c * dinv_r)

    m = jnp.concatenate(
        [jnp.dot(ans[i], feats_ref[i], preferred_element_type=jnp.float32)
         for i in range(GP)], axis=0)
    h = jnp.dot(m, w1_ref[...], preferred_element_type=jnp.float32)
    h = jnp.maximum(h + b1_ref[...], 0.0)

    p = jnp.concatenate(
        [jnp.dot(ans[i], h[i * N:(i + 1) * N, :],
                 preferred_element_type=jnp.float32)
         for i in range(GP)], axis=0)
    h2 = jnp.dot(p, w2_ref[...], preferred_element_type=jnp.float32)
    h2 = jnp.maximum(h2 + b2_ref[...], 0.0)
    for i in range(GP):
        pooled[pl.ds(step * GP + i, 1), :] = jnp.max(
            h2[i * N:(i + 1) * N, :], axis=0, keepdims=True)

    @pl.when(step == STEPS - 1)
    def _():
        xo = pooled[0:B, :]
        xm = pooled[B:G, :]
        t = (jnp.dot(xo, wt1_ref[0:D, :], preferred_element_type=jnp.float32)
             + jnp.dot(xm, wt1_ref[D:2 * D, :], preferred_element_type=jnp.float32)
             + bt1_ref[...])
        t = jnp.maximum(t, 0.0)
        out_ref[...] = (jnp.sum(t * wt2_ref[...], axis=1, keepdims=True)
                        + bt2_ref[...])


@functools.partial(jax.jit, static_argnames=("interpret",))
def kernel(coords_orig, feats_orig, coords_mut, feats_mut,
           W1, b1, W2, b2, Wt1, bt1, Wt2, bt2, interpret=False):
    coords = jnp.concatenate([coords_orig, coords_mut], axis=0)
    ca = jnp.pad(coords, ((0, 0), (0, 0), (0, 5)))
    cb = jnp.transpose(ca, (0, 2, 1))
    feats = jnp.concatenate([feats_orig, feats_mut], axis=0)

    const = lambda s: (0, 0)
    out = pl.pallas_call(
        _mspnet_kernel,
        grid=(STEPS,),
        in_specs=[
            pl.BlockSpec((GP, N, 8), lambda s: (s, 0, 0)),
            pl.BlockSpec((GP, 8, N), lambda s: (s, 0, 0)),
            pl.BlockSpec((GP, N, D), lambda s: (s, 0, 0)),
            pl.BlockSpec((D, D), const),
            pl.BlockSpec((1, D), const),
            pl.BlockSpec((D, D), const),
            pl.BlockSpec((1, D), const),
            pl.BlockSpec((2 * D, D), const),
            pl.BlockSpec((1, D), const),
            pl.BlockSpec((1, D), const),
            pl.BlockSpec((1, 1), const),
        ],
        out_specs=pl.BlockSpec((B, 1), const),
        out_shape=jax.ShapeDtypeStruct((B, 1), jnp.float32),
        scratch_shapes=[pltpu.VMEM((G, D), jnp.float32)],
        interpret=interpret,
    )(ca, cb, feats, W1, b1[None, :], W2, b2[None, :],
      Wt1, bt1[None, :], Wt2[:, 0][None, :], bt2.reshape(1, 1))
    return out


# speedup vs baseline: 3.3326x; 1.4424x over previous
import functools

import jax
import jax.numpy as jnp
from jax.experimental import pallas as pl
from jax.experimental.pallas import tpu as pltpu

B = 32
N = 128
D = 128
G = 2 * B
GP = 16
STEPS = G // GP
SIGMA = 2.5


def _mspnet_kernel(ca_ref, cb_ref, feats_ref, w1_ref, b1_ref, w2_ref, b2_ref,
                   wt1_ref, bt1_ref, wt2_ref, bt2_ref, out_ref, pooled):
    step = pl.program_id(0)

    row = jax.lax.broadcasted_iota(jnp.int32, (N, N), 0)
    col = jax.lax.broadcasted_iota(jnp.int32, (N, N), 1)

    ans = []
    for i in range(GP):
        ca = ca_ref[i]
        cb = cb_ref[i]
        gram = jnp.dot(ca, cb, preferred_element_type=jnp.float32)
        sq_c = jnp.sum(ca * ca, axis=1, keepdims=True)
        sq_r = jnp.sum(cb * cb, axis=0, keepdims=True)
        d2 = jnp.maximum(sq_c + sq_r - 2.0 * gram, 0.0)
        dist = jnp.sqrt(d2 + 1e-12)
        a = jnp.exp(dist * (-1.0 / SIGMA))
        a = jnp.where(row == col, 1.0, a)
        dinv_c = jax.lax.rsqrt(jnp.sum(a, axis=1, keepdims=True))
        dinv_r = jax.lax.rsqrt(jnp.sum(a, axis=0, keepdims=True))
        ans.append(a * dinv_c * dinv_r)

    m = jnp.concatenate(
        [jnp.dot(ans[i], feats_ref[i], preferred_element_type=jnp.float32)
         for i in range(GP)], axis=0)
    h = jnp.dot(m, w1_ref[...], preferred_element_type=jnp.float32)
    h = jnp.maximum(h + b1_ref[...], 0.0)

    p = jnp.concatenate(
        [jnp.dot(ans[i], h[i * N:(i + 1) * N, :],
                 preferred_element_type=jnp.float32)
         for i in range(GP)], axis=0)
    h2 = jnp.dot(p, w2_ref[...], preferred_element_type=jnp.float32)
    h2 = jnp.maximum(h2 + b2_ref[...], 0.0)
    pooled[pl.ds(step * GP, GP), :] = jnp.concatenate(
        [jnp.max(h2[i * N:(i + 1) * N, :], axis=0, keepdims=True)
         for i in range(GP)], axis=0)

    @pl.when(step == STEPS - 1)
    def _():
        xo = pooled[0:B, :]
        xm = pooled[B:G, :]
        t = (jnp.dot(xo, wt1_ref[0:D, :], preferred_element_type=jnp.float32)
             + jnp.dot(xm, wt1_ref[D:2 * D, :], preferred_element_type=jnp.float32)
             + bt1_ref[...])
        t = jnp.maximum(t, 0.0)
        out_ref[...] = (jnp.sum(t * wt2_ref[...], axis=1, keepdims=True)
                        + bt2_ref[...])


@functools.partial(jax.jit, static_argnames=("interpret",))
def kernel(coords_orig, feats_orig, coords_mut, feats_mut,
           W1, b1, W2, b2, Wt1, bt1, Wt2, bt2, interpret=False):
    coords = jnp.concatenate([coords_orig, coords_mut], axis=0)
    ca = jnp.pad(coords, ((0, 0), (0, 0), (0, 5)))
    cb = jnp.transpose(ca, (0, 2, 1))
    feats = jnp.concatenate([feats_orig, feats_mut], axis=0)

    const = lambda s: (0, 0)
    out = pl.pallas_call(
        _mspnet_kernel,
        grid=(STEPS,),
        in_specs=[
            pl.BlockSpec((GP, N, 8), lambda s: (s, 0, 0)),
            pl.BlockSpec((GP, 8, N), lambda s: (s, 0, 0)),
            pl.BlockSpec((GP, N, D), lambda s: (s, 0, 0)),
            pl.BlockSpec((D, D), const),
            pl.BlockSpec((1, D), const),
            pl.BlockSpec((D, D), const),
            pl.BlockSpec((1, D), const),
            pl.BlockSpec((2 * D, D), const),
            pl.BlockSpec((1, D), const),
            pl.BlockSpec((1, D), const),
            pl.BlockSpec((1, 1), const),
        ],
        out_specs=pl.BlockSpec((B, 1), const),
        out_shape=jax.ShapeDtypeStruct((B, 1), jnp.float32),
        scratch_shapes=[pltpu.VMEM((G, D), jnp.float32)],
        interpret=interpret,
    )(ca, cb, feats, W1, b1[None, :], W2, b2[None, :],
      Wt1, bt1[None, :], Wt2[:, 0][None, :], bt2.reshape(1, 1))
    return out
